# baseline (speedup 1.0000x reference)
import functools

import jax
import jax.numpy as jnp
from jax.experimental import pallas as pl
from jax.experimental.pallas import tpu as pltpu

ALPHA = 0.2


def _lrelu(v):
    return jnp.where(v > 0, v, ALPHA * v)


def _elu(v):
    return jnp.where(v > 0, v, jnp.exp(v) - 1.0)


def _proj_body(h_ref, w_ref, a_ref, b_ref, wh12_ref, mx_ref, *, fp, stride):
    i = pl.program_id(0)
    wh = jnp.dot(h_ref[...], w_ref[...], preferred_element_type=jnp.float32)
    wh12 = jnp.dot(wh, a_ref[...], preferred_element_type=jnp.float32)
    wh12_ref[...] = wh12
    nh = wh12.shape[1] // 2
    bm = jnp.max(wh12[:, nh:], axis=0, keepdims=True)

    @pl.when(i == 0)
    def _():
        mx_ref[...] = jnp.full(mx_ref.shape, -jnp.inf, mx_ref.dtype)

    mx_ref[0:1, 0:nh] = jnp.maximum(mx_ref[0:1, 0:nh], bm)

    half = stride // 2
    br = wh.shape[0]
    pad = jnp.zeros((br, half - fp - 1), jnp.float32)
    pieces = []
    for h in range(nh):
        whh = wh[:, h * fp:(h + 1) * fp]
        w2 = wh12[:, nh + h:nh + h + 1]
        v = jnp.exp(w2)
        v2 = jnp.exp(ALPHA * w2)
        pieces += [whh * v, v, pad, whh * v2, v2, pad]
    b_ref[...] = jnp.concatenate(pieces, axis=1).astype(jnp.bfloat16)


def _round_up(x, m):
    return (x + m - 1) // m * m


def _project(h, w, a, block_rows):
    n, f = h.shape
    nh2 = a.shape[1]
    nh = nh2 // 2
    fp = w.shape[1] // nh
    stride = 2 * _round_up(fp + 1, 128)
    grid = (n // block_rows,)
    body = functools.partial(_proj_body, fp=fp, stride=stride)
    b, wh12, mx = pl.pallas_call(
        body,
        grid=grid,
        in_specs=[
            pl.BlockSpec((block_rows, f), lambda i: (i, 0)),
            pl.BlockSpec((f, w.shape[1]), lambda i: (0, 0)),
            pl.BlockSpec((w.shape[1], nh2), lambda i: (0, 0)),
        ],
        out_specs=[
            pl.BlockSpec((block_rows, nh * stride), lambda i: (i, 0)),
            pl.BlockSpec((block_rows, nh2), lambda i: (i, 0)),
            pl.BlockSpec((8, 128), lambda i: (0, 0)),
        ],
        out_shape=[
            jax.ShapeDtypeStruct((n, nh * stride), jnp.bfloat16),
            jax.ShapeDtypeStruct((n, nh2), jnp.float32),
            jax.ShapeDtypeStruct((8, 128), jnp.float32),
        ],
        compiler_params=pltpu.CompilerParams(
            dimension_semantics=("arbitrary",),
        ),
    )(h, w, a)
    return b, wh12, mx, stride


def _attn_body(wh12i_ref, wh12t_ref, b_ref, mx_ref, adj_ref, out_ref,
               num_acc, den_acc, *, nheads, fp, bc, stride, mode):
    j = pl.program_id(1)
    nj = pl.num_programs(1)

    @pl.when(j == 0)
    def _():
        num_acc[...] = jnp.zeros(num_acc.shape, num_acc.dtype)
        den_acc[...] = jnp.zeros(den_acc.shape, den_acc.dtype)

    adj01 = adj_ref[...]
    col0 = j * bc
    half = stride // 2
    for h in range(nheads):
        wh1 = wh12i_ref[:, h:h + 1]
        wh2 = wh12t_ref[nheads + h:nheads + h + 1, pl.ds(col0, bc)]
        mx = mx_ref[0:1, h:h + 1]
        m = _lrelu(wh1 + mx)
        u = jnp.exp(wh1 - m)
        u2 = jnp.exp(ALPHA * wh1 - m)
        e = wh1 + wh2
        step = jnp.where(e > 0, 1.0, 0.0).astype(jnp.bfloat16)
        ap = adj01 * step
        am = adj01 - ap
        bpos = b_ref[pl.ds(col0, bc), h * stride:h * stride + fp + 1]
        bneg = b_ref[pl.ds(col0, bc), h * stride + half:h * stride + half + fp + 1]
        pp = jnp.dot(ap, bpos, preferred_element_type=jnp.float32)
        pm = jnp.dot(am, bneg, preferred_element_type=jnp.float32)
        comb = u * pp + u2 * pm
        num_acc[:, h * fp:(h + 1) * fp] += comb[:, :fp]
        den_acc[:, h:h + 1] += comb[:, fp:fp + 1]

    @pl.when(j == nj - 1)
    def _():
        if mode == "concat_elu":
            for h in range(nheads):
                v = num_acc[:, h * fp:(h + 1) * fp] / den_acc[:, h:h + 1]
                out_ref[:, h * fp:(h + 1) * fp] = _elu(_elu(v))
        else:
            v = num_acc[...] / den_acc[:, 0:1]
            vmax = jnp.max(v, axis=1, keepdims=True)
            vs = v - vmax
            lse = jnp.log(jnp.sum(jnp.exp(vs), axis=1, keepdims=True))
            out_ref[...] = vs - lse


def _attention(wh12, b, mx, adj, nheads, fp, stride, br, bc, mode):
    n = adj.shape[0]
    grid = (n // br, n // bc)
    wh12t = wh12.T
    body = functools.partial(_attn_body, nheads=nheads, fp=fp, bc=bc,
                             stride=stride, mode=mode)
    return pl.pallas_call(
        body,
        grid=grid,
        in_specs=[
            pl.BlockSpec((br, 2 * nheads), lambda i, j: (i, 0)),
            pl.BlockSpec((2 * nheads, n), lambda i, j: (0, 0)),
            pl.BlockSpec((n, nheads * stride), lambda i, j: (0, 0)),
            pl.BlockSpec((8, 128), lambda i, j: (0, 0)),
            pl.BlockSpec((br, bc), lambda i, j: (i, j)),
        ],
        out_specs=pl.BlockSpec((br, nheads * fp), lambda i, j: (i, 0)),
        out_shape=jax.ShapeDtypeStruct((n, nheads * fp), jnp.float32),
        scratch_shapes=[
            pltpu.VMEM((br, nheads * fp), jnp.float32),
            pltpu.VMEM((br, 128), jnp.float32),
        ],
        compiler_params=pltpu.CompilerParams(
            dimension_semantics=("arbitrary", "arbitrary"),
        ),
    )(wh12, wh12t, b, mx, adj)


def kernel(x, adj, W1, a1, W2, a2):
    n, nfeat = x.shape
    nheads, _, nhid = W1.shape
    nclass = W2.shape[1]

    w1cat = jnp.transpose(W1, (1, 0, 2)).reshape(nfeat, nheads * nhid)
    a1m = jnp.zeros((nheads * nhid, 2 * nheads), dtype=jnp.float32)
    for h in range(nheads):
        a1m = a1m.at[h * nhid:(h + 1) * nhid, h].set(a1[h, :nhid, 0])
        a1m = a1m.at[h * nhid:(h + 1) * nhid, nheads + h].set(a1[h, nhid:, 0])
    a2m = jnp.concatenate([a2[:nclass], a2[nclass:]], axis=1)

    br = min(256, n)
    bc = min(1024, n)
    adjb = adj.astype(jnp.bfloat16)

    b1, wh12, mx, stride1 = _project(x, w1cat, a1m, br)
    h1 = _attention(wh12, b1, mx, adjb, nheads, nhid, stride1, br, bc,
                    "concat_elu")

    b2, wh12p, mxp, stride2 = _project(h1, W2, a2m, br)
    out = _attention(wh12p, b2, mxp, adjb, 1, nclass, stride2, br, bc,
                     "log_softmax")
    return out


# speedup vs baseline: 1.1547x; 1.1547x over previous
import functools

import jax
import jax.numpy as jnp
from jax.experimental import pallas as pl
from jax.experimental.pallas import tpu as pltpu

ALPHA = 0.2


def _lrelu(v):
    return jnp.where(v > 0, v, ALPHA * v)


def _elu(v):
    return jnp.where(v > 0, v, jnp.exp(v) - 1.0)


def _proj_body(h_ref, w_ref, a_ref, b_ref, wh12_ref, mx_ref, *, fp, stride):
    i = pl.program_id(0)
    wh = jnp.dot(h_ref[...], w_ref[...], preferred_element_type=jnp.float32)
    wh12 = jnp.dot(wh, a_ref[...], preferred_element_type=jnp.float32)
    wh12_ref[...] = wh12
    nh = wh12.shape[1] // 2
    bm = jnp.max(wh12[:, nh:], axis=0, keepdims=True)

    @pl.when(i == 0)
    def _():
        mx_ref[...] = jnp.full(mx_ref.shape, -jnp.inf, mx_ref.dtype)

    mx_ref[0:1, 0:nh] = jnp.maximum(mx_ref[0:1, 0:nh], bm)

    half = stride // 2
    br = wh.shape[0]
    pad = jnp.zeros((br, half - fp - 1), jnp.float32)
    pieces = []
    for h in range(nh):
        whh = wh[:, h * fp:(h + 1) * fp]
        w2 = wh12[:, nh + h:nh + h + 1]
        v = jnp.exp(w2)
        v2 = jnp.exp(ALPHA * w2)
        pieces += [whh * v, v, pad, whh * v2, v2, pad]
    b_ref[...] = jnp.concatenate(pieces, axis=1).astype(jnp.bfloat16)


def _round_up(x, m):
    return (x + m - 1) // m * m


def _project(h, w, a, block_rows):
    n, f = h.shape
    nh2 = a.shape[1]
    nh = nh2 // 2
    fp = w.shape[1] // nh
    stride = 2 * _round_up(fp + 1, 128)
    grid = (n // block_rows,)
    body = functools.partial(_proj_body, fp=fp, stride=stride)
    b, wh12, mx = pl.pallas_call(
        body,
        grid=grid,
        in_specs=[
            pl.BlockSpec((block_rows, f), lambda i: (i, 0)),
            pl.BlockSpec((f, w.shape[1]), lambda i: (0, 0)),
            pl.BlockSpec((w.shape[1], nh2), lambda i: (0, 0)),
        ],
        out_specs=[
            pl.BlockSpec((block_rows, nh * stride), lambda i: (i, 0)),
            pl.BlockSpec((block_rows, nh2), lambda i: (i, 0)),
            pl.BlockSpec((8, 128), lambda i: (0, 0)),
        ],
        out_shape=[
            jax.ShapeDtypeStruct((n, nh * stride), jnp.bfloat16),
            jax.ShapeDtypeStruct((n, nh2), jnp.float32),
            jax.ShapeDtypeStruct((8, 128), jnp.float32),
        ],
        compiler_params=pltpu.CompilerParams(
            dimension_semantics=("arbitrary",),
        ),
    )(h, w, a)
    return b, wh12, mx, stride


def _attn_body(wh12i_ref, wh12t_ref, b_ref, mx_ref, adj_ref, out_ref,
               num_acc, den_acc, *, nheads, fp, bc, stride, mode):
    j = pl.program_id(1)
    nj = pl.num_programs(1)

    @pl.when(j == 0)
    def _():
        num_acc[...] = jnp.zeros(num_acc.shape, num_acc.dtype)
        den_acc[...] = jnp.zeros(den_acc.shape, den_acc.dtype)

    adj01 = adj_ref[...].astype(jnp.bfloat16)
    col0 = j * bc
    half = stride // 2
    for h in range(nheads):
        wh1 = wh12i_ref[:, h:h + 1]
        wh2 = wh12t_ref[nheads + h:nheads + h + 1, pl.ds(col0, bc)]
        mx = mx_ref[0:1, h:h + 1]
        m = _lrelu(wh1 + mx)
        u = jnp.exp(wh1 - m)
        u2 = jnp.exp(ALPHA * wh1 - m)
        wh1b = wh1.astype(jnp.bfloat16)
        wh2b = wh2.astype(jnp.bfloat16)
        e = wh1b + wh2b
        ap = jnp.where(e > 0, adj01, jnp.bfloat16(0))
        am = adj01 - ap
        bpos = b_ref[pl.ds(col0, bc), h * stride:h * stride + fp + 1]
        bneg = b_ref[pl.ds(col0, bc), h * stride + half:h * stride + half + fp + 1]
        pp = jnp.dot(ap, bpos, preferred_element_type=jnp.float32)
        pm = jnp.dot(am, bneg, preferred_element_type=jnp.float32)
        comb = u * pp + u2 * pm
        num_acc[:, h * fp:(h + 1) * fp] += comb[:, :fp]
        den_acc[:, h:h + 1] += comb[:, fp:fp + 1]

    @pl.when(j == nj - 1)
    def _():
        if mode == "concat_elu":
            for h in range(nheads):
                v = num_acc[:, h * fp:(h + 1) * fp] / den_acc[:, h:h + 1]
                out_ref[:, h * fp:(h + 1) * fp] = _elu(_elu(v))
        else:
            v = num_acc[...] / den_acc[:, 0:1]
            vmax = jnp.max(v, axis=1, keepdims=True)
            vs = v - vmax
            lse = jnp.log(jnp.sum(jnp.exp(vs), axis=1, keepdims=True))
            out_ref[...] = vs - lse


def _attention(wh12, b, mx, adj, nheads, fp, stride, br, bc, mode):
    n = adj.shape[0]
    grid = (n // br, n // bc)
    wh12t = wh12.T
    body = functools.partial(_attn_body, nheads=nheads, fp=fp, bc=bc,
                             stride=stride, mode=mode)
    return pl.pallas_call(
        body,
        grid=grid,
        in_specs=[
            pl.BlockSpec((br, 2 * nheads), lambda i, j: (i, 0)),
            pl.BlockSpec((2 * nheads, n), lambda i, j: (0, 0)),
            pl.BlockSpec((n, nheads * stride), lambda i, j: (0, 0)),
            pl.BlockSpec((8, 128), lambda i, j: (0, 0)),
            pl.BlockSpec((br, bc), lambda i, j: (i, j)),
        ],
        out_specs=pl.BlockSpec((br, nheads * fp), lambda i, j: (i, 0)),
        out_shape=jax.ShapeDtypeStruct((n, nheads * fp), jnp.float32),
        scratch_shapes=[
            pltpu.VMEM((br, nheads * fp), jnp.float32),
            pltpu.VMEM((br, 128), jnp.float32),
        ],
        compiler_params=pltpu.CompilerParams(
            dimension_semantics=("arbitrary", "arbitrary"),
        ),
    )(wh12, wh12t, b, mx, adj)


def kernel(x, adj, W1, a1, W2, a2):
    n, nfeat = x.shape
    nheads, _, nhid = W1.shape
    nclass = W2.shape[1]

    w1cat = jnp.transpose(W1, (1, 0, 2)).reshape(nfeat, nheads * nhid)
    a1m = jnp.zeros((nheads * nhid, 2 * nheads), dtype=jnp.float32)
    for h in range(nheads):
        a1m = a1m.at[h * nhid:(h + 1) * nhid, h].set(a1[h, :nhid, 0])
        a1m = a1m.at[h * nhid:(h + 1) * nhid, nheads + h].set(a1[h, nhid:, 0])
    a2m = jnp.concatenate([a2[:nclass], a2[nclass:]], axis=1)

    br = min(256, n)
    bc = min(1024, n)
    b1, wh12, mx, stride1 = _project(x, w1cat, a1m, br)
    h1 = _attention(wh12, b1, mx, adj, nheads, nhid, stride1, br, bc,
                    "concat_elu")

    b2, wh12p, mxp, stride2 = _project(h1, W2, a2m, br)
    out = _attention(wh12p, b2, mxp, adj, 1, nclass, stride2, br, bc,
                     "log_softmax")
    return out


# speedup vs baseline: 1.1663x; 1.0101x over previous
import functools

import jax
import jax.numpy as jnp
from jax.experimental import pallas as pl
from jax.experimental.pallas import tpu as pltpu

ALPHA = 0.2


def _lrelu(v):
    return jnp.where(v > 0, v, ALPHA * v)


def _elu(v):
    return jnp.where(v > 0, v, jnp.exp(v) - 1.0)


def _proj_body(h_ref, w_ref, a_ref, b_ref, wh12_ref, mx_ref, *, fp, stride):
    i = pl.program_id(0)
    wh = jnp.dot(h_ref[...], w_ref[...], preferred_element_type=jnp.float32)
    wh12 = jnp.dot(wh, a_ref[...], preferred_element_type=jnp.float32)
    wh12_ref[...] = wh12
    nh = wh12.shape[1] // 2
    bm = jnp.max(wh12[:, nh:], axis=0, keepdims=True)

    @pl.when(i == 0)
    def _():
        mx_ref[...] = jnp.full(mx_ref.shape, -jnp.inf, mx_ref.dtype)

    mx_ref[0:1, 0:nh] = jnp.maximum(mx_ref[0:1, 0:nh], bm)

    half = stride // 2
    br = wh.shape[0]
    pad = jnp.zeros((br, half - fp - 1), jnp.float32)
    pieces = []
    for h in range(nh):
        whh = wh[:, h * fp:(h + 1) * fp]
        w2 = wh12[:, nh + h:nh + h + 1]
        v = jnp.exp(w2)
        v2 = jnp.exp(ALPHA * w2)
        pieces += [whh * v, v, pad, whh * v2, v2, pad]
    b_ref[...] = jnp.concatenate(pieces, axis=1).astype(jnp.bfloat16)


def _round_up(x, m):
    return (x + m - 1) // m * m


def _project(h, w, a, block_rows):
    n, f = h.shape
    nh2 = a.shape[1]
    nh = nh2 // 2
    fp = w.shape[1] // nh
    stride = 2 * _round_up(fp + 1, 128)
    grid = (n // block_rows,)
    body = functools.partial(_proj_body, fp=fp, stride=stride)
    b, wh12, mx = pl.pallas_call(
        body,
        grid=grid,
        in_specs=[
            pl.BlockSpec((block_rows, f), lambda i: (i, 0)),
            pl.BlockSpec((f, w.shape[1]), lambda i: (0, 0)),
            pl.BlockSpec((w.shape[1], nh2), lambda i: (0, 0)),
        ],
        out_specs=[
            pl.BlockSpec((block_rows, nh * stride), lambda i: (i, 0)),
            pl.BlockSpec((block_rows, nh2), lambda i: (i, 0)),
            pl.BlockSpec((8, 128), lambda i: (0, 0)),
        ],
        out_shape=[
            jax.ShapeDtypeStruct((n, nh * stride), jnp.bfloat16),
            jax.ShapeDtypeStruct((n, nh2), jnp.float32),
            jax.ShapeDtypeStruct((8, 128), jnp.float32),
        ],
        compiler_params=pltpu.CompilerParams(
            dimension_semantics=("arbitrary",),
        ),
    )(h, w, a)
    return b, wh12, mx, stride


def _attn_body(wh12i_ref, wh12t_ref, b_ref, mx_ref, adj_ref, out_ref,
               accp_ref, accm_ref, *, nheads, fp, bc, stride, mode):
    j = pl.program_id(1)
    nj = pl.num_programs(1)

    @pl.when(j == 0)
    def _():
        accp_ref[...] = jnp.zeros(accp_ref.shape, accp_ref.dtype)
        accm_ref[...] = jnp.zeros(accm_ref.shape, accm_ref.dtype)

    adj01 = adj_ref[...].astype(jnp.bfloat16)
    col0 = j * bc
    half = stride // 2
    fp1 = fp + 1
    for h in range(nheads):
        wh1 = wh12i_ref[:, h:h + 1]
        wh2 = wh12t_ref[nheads + h:nheads + h + 1, pl.ds(col0, bc)]
        wh1b = wh1.astype(jnp.bfloat16)
        wh2b = wh2.astype(jnp.bfloat16)
        e = wh1b + wh2b
        ap = jnp.where(e > 0, adj01, jnp.bfloat16(0))
        am = adj01 - ap
        bpos = b_ref[pl.ds(col0, bc), h * stride:h * stride + fp1]
        bneg = b_ref[pl.ds(col0, bc), h * stride + half:h * stride + half + fp1]
        pp = jnp.dot(ap, bpos, preferred_element_type=jnp.float32)
        pm = jnp.dot(am, bneg, preferred_element_type=jnp.float32)
        accp_ref[:, h * fp1:(h + 1) * fp1] += pp
        accm_ref[:, h * fp1:(h + 1) * fp1] += pm

    @pl.when(j == nj - 1)
    def _():
        for h in range(nheads):
            wh1 = wh12i_ref[:, h:h + 1]
            mx = mx_ref[0:1, h:h + 1]
            m = _lrelu(wh1 + mx)
            u = jnp.exp(wh1 - m)
            u2 = jnp.exp(ALPHA * wh1 - m)
            comb = (u * accp_ref[:, h * fp1:(h + 1) * fp1]
                    + u2 * accm_ref[:, h * fp1:(h + 1) * fp1])
            v = comb[:, :fp] / comb[:, fp:fp1]
            if mode == "concat_elu":
                out_ref[:, h * fp:(h + 1) * fp] = _elu(_elu(v))
            else:
                vmax = jnp.max(v, axis=1, keepdims=True)
                vs = v - vmax
                lse = jnp.log(jnp.sum(jnp.exp(vs), axis=1, keepdims=True))
                out_ref[...] = vs - lse


def _attention(wh12, b, mx, adj, nheads, fp, stride, br, bc, mode):
    n = adj.shape[0]
    grid = (n // br, n // bc)
    wh12t = wh12.T
    body = functools.partial(_attn_body, nheads=nheads, fp=fp, bc=bc,
                             stride=stride, mode=mode)
    return pl.pallas_call(
        body,
        grid=grid,
        in_specs=[
            pl.BlockSpec((br, 2 * nheads), lambda i, j: (i, 0)),
            pl.BlockSpec((2 * nheads, n), lambda i, j: (0, 0)),
            pl.BlockSpec((n, nheads * stride), lambda i, j: (0, 0)),
            pl.BlockSpec((8, 128), lambda i, j: (0, 0)),
            pl.BlockSpec((br, bc), lambda i, j: (i, j)),
        ],
        out_specs=pl.BlockSpec((br, nheads * fp), lambda i, j: (i, 0)),
        out_shape=jax.ShapeDtypeStruct((n, nheads * fp), jnp.float32),
        scratch_shapes=[
            pltpu.VMEM((br, nheads * (fp + 1)), jnp.float32),
            pltpu.VMEM((br, nheads * (fp + 1)), jnp.float32),
        ],
        compiler_params=pltpu.CompilerParams(
            dimension_semantics=("arbitrary", "arbitrary"),
        ),
    )(wh12, wh12t, b, mx, adj)


def kernel(x, adj, W1, a1, W2, a2):
    n, nfeat = x.shape
    nheads, _, nhid = W1.shape
    nclass = W2.shape[1]

    w1cat = jnp.transpose(W1, (1, 0, 2)).reshape(nfeat, nheads * nhid)
    a1m = jnp.zeros((nheads * nhid, 2 * nheads), dtype=jnp.float32)
    for h in range(nheads):
        a1m = a1m.at[h * nhid:(h + 1) * nhid, h].set(a1[h, :nhid, 0])
        a1m = a1m.at[h * nhid:(h + 1) * nhid, nheads + h].set(a1[h, nhid:, 0])
    a2m = jnp.concatenate([a2[:nclass], a2[nclass:]], axis=1)

    br = min(256, n)
    bc = min(1024, n)
    b1, wh12, mx, stride1 = _project(x, w1cat, a1m, br)
    h1 = _attention(wh12, b1, mx, adj, nheads, nhid, stride1, br, bc,
                    "concat_elu")

    b2, wh12p, mxp, stride2 = _project(h1, W2, a2m, br)
    out = _attention(wh12p, b2, mxp, adj, 1, nclass, stride2, br, bc,
                     "log_softmax")
    return out
